# baseline (speedup 1.0000x reference)
import functools

import jax
import jax.numpy as jnp
from jax import lax
from jax.experimental import pallas as pl
from jax.experimental.pallas import tpu as pltpu
from jax.experimental.pallas import tpu_sc as plsc

_N = 10000
_D = 128
_E = 320000
_G = 64

_NC = 2
_NS = 16
_CHUNK = 128
_TOTCH = 2560
_KPT = _TOTCH // _NS
_GRP = 8
_EPAD = _TOTCH * _CHUNK
_RPAD = 10240
_RPT = _RPAD // _NS
_BLK = 512


@functools.lru_cache(maxsize=None)
def _edge_agg_kernel():
    return functools.partial(
        pl.kernel,
        out_type=jax.ShapeDtypeStruct((_RPAD, _D), jnp.float32),
        mesh=plsc.VectorSubcoreMesh(
            core_axis_name="c", subcore_axis_name="s", num_cores=1, num_subcores=_NS
        ),
        scratch_types=[
            pltpu.VMEM((_GRP, _CHUNK), jnp.int32),
            pltpu.VMEM((_GRP, _CHUNK), jnp.int32),
            pltpu.VMEM((_CHUNK, _D), jnp.float32),
            pltpu.VMEM((_CHUNK, _D), jnp.float32),
            pltpu.VMEM_SHARED((_RPAD, _D), jnp.float32),
            pltpu.SemaphoreType.DMA,
            pltpu.SemaphoreType.DMA,
        ],
    )(_edge_agg_body)


def _edge_agg_body(feat, srcs, dsts, out, src_v, dst_v, rows_a, rows_b, acc, sem_a, sem_b):
    sid = lax.axis_index("s")
    base = sid * _RPT

    def zrow(i, carry):
        rows_a[lax.div(i, 8), pl.ds(lax.rem(i, 8) * 16, 16)] = jnp.zeros(
            (16,), jnp.float32)
        return carry

    lax.fori_loop(0, _CHUNK * 8, zrow, 0)
    for r in range(_RPT // _CHUNK):
        pltpu.sync_copy(rows_a, acc.at[pl.ds(base + r * _CHUNK, _CHUNK)])
    plsc.subcore_barrier()

    def run_chunks(tile0, ngrp):
        for g in range(ngrp):
            goff = tile0 + g * _GRP
            pltpu.sync_copy(srcs.at[pl.ds(goff, _GRP)], src_v)
            pltpu.sync_copy(dsts.at[pl.ds(goff, _GRP)], dst_v)
            pltpu.async_copy(feat.at[src_v.at[0]], rows_a, sem_a)

            def chunk(j, carry):
                @pl.when(lax.rem(j, 2) == 0)
                def _():
                    @pl.when(j + 1 < _GRP)
                    def _():
                        pltpu.async_copy(feat.at[src_v.at[j + 1]], rows_b, sem_b)
                    pltpu.make_async_copy(feat.at[src_v.at[j]], rows_a, sem_a).wait()
                    pltpu.sync_copy(rows_a, acc.at[dst_v.at[j]], add=True)

                @pl.when(lax.rem(j, 2) == 1)
                def _():
                    @pl.when(j + 1 < _GRP)
                    def _():
                        pltpu.async_copy(feat.at[src_v.at[j + 1]], rows_a, sem_a)
                    pltpu.make_async_copy(feat.at[src_v.at[j]], rows_b, sem_b).wait()
                    pltpu.sync_copy(rows_b, acc.at[dst_v.at[j]], add=True)

                return carry

            lax.fori_loop(0, _GRP, chunk, 0)

    run_chunks(sid * _KPT, _KPT // _GRP)

    plsc.subcore_barrier()
    pltpu.sync_copy(acc.at[pl.ds(base, _RPT)], out.at[pl.ds(base, _RPT)])


def _dot_t(a, w):
    return lax.dot_general(a, w, (((1,), (1,)), ((), ())),
                           preferred_element_type=jnp.float32)


def _conv_body(agg_ref, feat_ref, wl_ref, bl_ref, wr_ref, out_ref):
    a = agg_ref[...]
    h = _dot_t(a, wl_ref[...]) + bl_ref[...] + _dot_t(feat_ref[...], wr_ref[...])
    out_ref[...] = jnp.maximum(h, 0.0)


_conv_tc = pl.pallas_call(
    _conv_body,
    grid=(_RPAD // _BLK,),
    in_specs=[
        pl.BlockSpec((_BLK, _D), lambda i: (i, 0)),
        pl.BlockSpec((_BLK, _D), lambda i: (i, 0)),
        pl.BlockSpec((_D, _D), lambda i: (0, 0)),
        pl.BlockSpec((1, _D), lambda i: (0, 0)),
        pl.BlockSpec((_D, _D), lambda i: (0, 0)),
    ],
    out_specs=pl.BlockSpec((_BLK, _D), lambda i: (i, 0)),
    out_shape=jax.ShapeDtypeStruct((_RPAD, _D), jnp.float32),
)


def _conv_pool_body(agg_ref, feat_ref, batch_ref, wl_ref, bl_ref, wr_ref,
                    w1_ref, b1_ref, w2_ref, b2_ref, w3_ref, b3_ref,
                    out_ref, pooled):
    i = pl.program_id(0)
    a = agg_ref[...]
    h = jnp.maximum(
        _dot_t(a, wl_ref[...]) + bl_ref[...] + _dot_t(feat_ref[...], wr_ref[...]),
        0.0,
    )
    bb = batch_ref[0, 0, :]
    onehot = (bb[None, :] == lax.broadcasted_iota(jnp.int32, (_G, _BLK), 0)
              ).astype(jnp.float32)
    contrib = jnp.dot(onehot, h, preferred_element_type=jnp.float32)

    @pl.when(i == 0)
    def _():
        pooled[...] = contrib

    @pl.when(i > 0)
    def _():
        pooled[...] = pooled[...] + contrib

    @pl.when(i == pl.num_programs(0) - 1)
    def _():
        z = jnp.maximum(_dot_t(pooled[...], w1_ref[...]) + b1_ref[...], 0.0)
        z = jnp.maximum(_dot_t(z, w2_ref[...]) + b2_ref[...], 0.0)
        out_ref[...] = _dot_t(z, w3_ref[...]) + b3_ref[...]


_conv_pool_tc = pl.pallas_call(
    _conv_pool_body,
    grid=(_RPAD // _BLK,),
    in_specs=[
        pl.BlockSpec((_BLK, _D), lambda i: (i, 0)),
        pl.BlockSpec((_BLK, _D), lambda i: (i, 0)),
        pl.BlockSpec((1, 1, _BLK), lambda i: (i, 0, 0)),
        pl.BlockSpec((_D, _D), lambda i: (0, 0)),
        pl.BlockSpec((1, _D), lambda i: (0, 0)),
        pl.BlockSpec((_D, _D), lambda i: (0, 0)),
        pl.BlockSpec((64, _D), lambda i: (0, 0)),
        pl.BlockSpec((1, 64), lambda i: (0, 0)),
        pl.BlockSpec((32, 64), lambda i: (0, 0)),
        pl.BlockSpec((1, 32), lambda i: (0, 0)),
        pl.BlockSpec((10, 32), lambda i: (0, 0)),
        pl.BlockSpec((1, 10), lambda i: (0, 0)),
    ],
    out_specs=pl.BlockSpec((_G, 10), lambda i: (0, 0)),
    out_shape=jax.ShapeDtypeStruct((_G, 10), jnp.float32),
    scratch_shapes=[pltpu.VMEM((_G, _D), jnp.float32)],
)


def kernel(x, edge_index, batch, Wl1, bl1, Wr1, Wl2, bl2, Wr2, W1, b1, W2, b2, W3, b3):
    src = edge_index[0]
    dst = edge_index[1]
    pad_e = _EPAD - _E
    srcs = jnp.concatenate([src, jnp.zeros((pad_e,), jnp.int32)]).reshape(_TOTCH, _CHUNK)
    dsts = jnp.concatenate([dst, jnp.full((pad_e,), _N, jnp.int32)]).reshape(_TOTCH, _CHUNK)
    xp = jnp.pad(x, ((0, _RPAD - _N), (0, 0)))
    batch_r = jnp.pad(batch, (0, _RPAD - _N), constant_values=_G).reshape(
        _RPAD // _BLK, 1, _BLK)

    edge_agg = _edge_agg_kernel()
    agg1 = edge_agg(xp, srcs, dsts)
    h1 = _conv_tc(agg1, xp, Wl1, bl1.reshape(1, -1), Wr1)
    agg2 = edge_agg(h1, srcs, dsts)
    out = _conv_pool_tc(agg2, h1, batch_r, Wl2, bl2.reshape(1, -1), Wr2,
                        W1, b1.reshape(1, -1), W2, b2.reshape(1, -1),
                        W3, b3.reshape(1, -1))
    return out


# speedup vs baseline: 1.1959x; 1.1959x over previous
import functools

import jax
import jax.numpy as jnp
from jax import lax
from jax.experimental import pallas as pl
from jax.experimental.pallas import tpu as pltpu
from jax.experimental.pallas import tpu_sc as plsc

_N = 10000
_D = 128
_E = 320000
_G = 64

_NC = 2
_NS = 16
_CHUNK = 128
_TOTCH = 2560
_K0 = 80
_K1 = (_TOTCH // _NS) - _K0
_GRP = 16
_EPAD = _TOTCH * _CHUNK
_RPAD = 10240
_RPT = _RPAD // _NS
_BLK = 512


@functools.lru_cache(maxsize=None)
def _edge_agg_kernel():
    return functools.partial(
        pl.kernel,
        out_type=jax.ShapeDtypeStruct((_NC, _RPAD, _D), jnp.float32),
        mesh=plsc.VectorSubcoreMesh(
            core_axis_name="c", subcore_axis_name="s", num_cores=_NC, num_subcores=_NS
        ),
        scratch_types=[
            pltpu.VMEM((_GRP, _CHUNK), jnp.int32),
            pltpu.VMEM((_GRP, _CHUNK), jnp.int32),
            pltpu.VMEM((_CHUNK, _D), jnp.float32),
            pltpu.VMEM((_CHUNK, _D), jnp.float32),
            pltpu.VMEM_SHARED((_RPAD, _D), jnp.float32),
            pltpu.SemaphoreType.DMA,
            pltpu.SemaphoreType.DMA,
        ],
    )(_edge_agg_body)


def _edge_agg_body(feat, srcs, dsts, out, src_v, dst_v, rows_a, rows_b, acc, sem_a, sem_b):
    cid = lax.axis_index("c")
    sid = lax.axis_index("s")
    base = sid * _RPT

    def zrow(i, carry):
        rows_a[lax.div(i, 8), pl.ds(lax.rem(i, 8) * 16, 16)] = jnp.zeros(
            (16,), jnp.float32)
        return carry

    lax.fori_loop(0, _CHUNK * 8, zrow, 0)
    for r in range(_RPT // _CHUNK):
        pltpu.sync_copy(rows_a, acc.at[pl.ds(base + r * _CHUNK, _CHUNK)])
    plsc.subcore_barrier()

    def run_chunks(tile0, ngrp):
        for g in range(ngrp):
            goff = tile0 + g * _GRP
            pltpu.sync_copy(srcs.at[pl.ds(goff, _GRP)], src_v)
            pltpu.sync_copy(dsts.at[pl.ds(goff, _GRP)], dst_v)
            pltpu.async_copy(feat.at[src_v.at[0]], rows_a, sem_a)

            def chunk(j, carry):
                @pl.when(lax.rem(j, 2) == 0)
                def _():
                    @pl.when(j + 1 < _GRP)
                    def _():
                        pltpu.async_copy(feat.at[src_v.at[j + 1]], rows_b, sem_b)
                    pltpu.make_async_copy(feat.at[src_v.at[j]], rows_a, sem_a).wait()
                    pltpu.sync_copy(rows_a, acc.at[dst_v.at[j]], add=True)

                @pl.when(lax.rem(j, 2) == 1)
                def _():
                    @pl.when(j + 1 < _GRP)
                    def _():
                        pltpu.async_copy(feat.at[src_v.at[j + 1]], rows_a, sem_a)
                    pltpu.make_async_copy(feat.at[src_v.at[j]], rows_b, sem_b).wait()
                    pltpu.sync_copy(rows_b, acc.at[dst_v.at[j]], add=True)

                return carry

            lax.fori_loop(0, _GRP, chunk, 0)

    @pl.when(cid == 0)
    def _():
        run_chunks(sid * _K0, _K0 // _GRP)

    @pl.when(cid == 1)
    def _():
        run_chunks(_NS * _K0 + sid * _K1, _K1 // _GRP)

    plsc.subcore_barrier()
    pltpu.sync_copy(acc.at[pl.ds(base, _RPT)], out.at[cid, pl.ds(base, _RPT)])


def _dot_t(a, w):
    return lax.dot_general(a, w, (((1,), (1,)), ((), ())),
                           preferred_element_type=jnp.float32)


def _conv_body(agg_ref, feat_ref, wl_ref, bl_ref, wr_ref, out_ref):
    a = agg_ref[0] + agg_ref[1]
    h = _dot_t(a, wl_ref[...]) + bl_ref[...] + _dot_t(feat_ref[...], wr_ref[...])
    out_ref[...] = jnp.maximum(h, 0.0)


_conv_tc = pl.pallas_call(
    _conv_body,
    grid=(_RPAD // _BLK,),
    in_specs=[
        pl.BlockSpec((_NC, _BLK, _D), lambda i: (0, i, 0)),
        pl.BlockSpec((_BLK, _D), lambda i: (i, 0)),
        pl.BlockSpec((_D, _D), lambda i: (0, 0)),
        pl.BlockSpec((1, _D), lambda i: (0, 0)),
        pl.BlockSpec((_D, _D), lambda i: (0, 0)),
    ],
    out_specs=pl.BlockSpec((_BLK, _D), lambda i: (i, 0)),
    out_shape=jax.ShapeDtypeStruct((_RPAD, _D), jnp.float32),
)


def _conv_pool_body(agg_ref, feat_ref, batch_ref, wl_ref, bl_ref, wr_ref,
                    w1_ref, b1_ref, w2_ref, b2_ref, w3_ref, b3_ref,
                    out_ref, pooled):
    i = pl.program_id(0)
    a = agg_ref[0] + agg_ref[1]
    h = jnp.maximum(
        _dot_t(a, wl_ref[...]) + bl_ref[...] + _dot_t(feat_ref[...], wr_ref[...]),
        0.0,
    )
    bb = batch_ref[0, 0, :]
    onehot = (bb[None, :] == lax.broadcasted_iota(jnp.int32, (_G, _BLK), 0)
              ).astype(jnp.float32)
    contrib = jnp.dot(onehot, h, preferred_element_type=jnp.float32)

    @pl.when(i == 0)
    def _():
        pooled[...] = contrib

    @pl.when(i > 0)
    def _():
        pooled[...] = pooled[...] + contrib

    @pl.when(i == pl.num_programs(0) - 1)
    def _():
        z = jnp.maximum(_dot_t(pooled[...], w1_ref[...]) + b1_ref[...], 0.0)
        z = jnp.maximum(_dot_t(z, w2_ref[...]) + b2_ref[...], 0.0)
        out_ref[...] = _dot_t(z, w3_ref[...]) + b3_ref[...]


_conv_pool_tc = pl.pallas_call(
    _conv_pool_body,
    grid=(_RPAD // _BLK,),
    in_specs=[
        pl.BlockSpec((_NC, _BLK, _D), lambda i: (0, i, 0)),
        pl.BlockSpec((_BLK, _D), lambda i: (i, 0)),
        pl.BlockSpec((1, 1, _BLK), lambda i: (i, 0, 0)),
        pl.BlockSpec((_D, _D), lambda i: (0, 0)),
        pl.BlockSpec((1, _D), lambda i: (0, 0)),
        pl.BlockSpec((_D, _D), lambda i: (0, 0)),
        pl.BlockSpec((64, _D), lambda i: (0, 0)),
        pl.BlockSpec((1, 64), lambda i: (0, 0)),
        pl.BlockSpec((32, 64), lambda i: (0, 0)),
        pl.BlockSpec((1, 32), lambda i: (0, 0)),
        pl.BlockSpec((10, 32), lambda i: (0, 0)),
        pl.BlockSpec((1, 10), lambda i: (0, 0)),
    ],
    out_specs=pl.BlockSpec((_G, 10), lambda i: (0, 0)),
    out_shape=jax.ShapeDtypeStruct((_G, 10), jnp.float32),
    scratch_shapes=[pltpu.VMEM((_G, _D), jnp.float32)],
)


def kernel(x, edge_index, batch, Wl1, bl1, Wr1, Wl2, bl2, Wr2, W1, b1, W2, b2, W3, b3):
    src = edge_index[0]
    dst = edge_index[1]
    pad_e = _EPAD - _E
    srcs = jnp.concatenate([src, jnp.zeros((pad_e,), jnp.int32)]).reshape(_TOTCH, _CHUNK)
    pad_dst = _N + (jnp.arange(pad_e, dtype=jnp.int32) % (_RPAD - _N))
    dsts = jnp.concatenate([dst, pad_dst]).reshape(_TOTCH, _CHUNK)
    xp = jnp.pad(x, ((0, _RPAD - _N), (0, 0)))
    batch_r = jnp.pad(batch, (0, _RPAD - _N), constant_values=_G).reshape(
        _RPAD // _BLK, 1, _BLK)

    edge_agg = _edge_agg_kernel()
    agg1 = edge_agg(xp, srcs, dsts)
    h1 = _conv_tc(agg1, xp, Wl1, bl1.reshape(1, -1), Wr1)
    agg2 = edge_agg(h1, srcs, dsts)
    out = _conv_pool_tc(agg2, h1, batch_r, Wl2, bl2.reshape(1, -1), Wr2,
                        W1, b1.reshape(1, -1), W2, b2.reshape(1, -1),
                        W3, b3.reshape(1, -1))
    return out


# speedup vs baseline: 3.7206x; 3.1112x over previous
import functools

import jax
import jax.numpy as jnp
from jax import lax
from jax.experimental import pallas as pl
from jax.experimental.pallas import tpu as pltpu
from jax.experimental.pallas import tpu_sc as plsc

_N = 10000
_D = 128
_E = 320000
_G = 64

_NC = 2
_NS = 16
_CHUNK = 128
_TOTCH = 2560
_K0 = 80
_K1 = (_TOTCH // _NS) - _K0
_GRP = 16
_EPAD = _TOTCH * _CHUNK
_RPAD = 10240
_RPT = _RPAD // _NS
_BLK = 512


@functools.lru_cache(maxsize=None)
def _edge_agg_kernel():
    return functools.partial(
        pl.kernel,
        out_type=jax.ShapeDtypeStruct((_NC, _RPAD, _D), jnp.float32),
        mesh=plsc.VectorSubcoreMesh(
            core_axis_name="c", subcore_axis_name="s", num_cores=_NC, num_subcores=_NS
        ),
        scratch_types=[
            pltpu.VMEM((_GRP, _CHUNK), jnp.int32),
            pltpu.VMEM((_GRP, _CHUNK), jnp.int32),
            pltpu.VMEM((_CHUNK, _D), jnp.float32),
            pltpu.VMEM((_CHUNK, _D), jnp.float32),
            pltpu.VMEM_SHARED((_RPAD, _D), jnp.float32),
            pltpu.SemaphoreType.DMA,
            pltpu.SemaphoreType.DMA,
        ],
    )(_edge_agg_body)


def _edge_agg_body(feat, srcs, dsts, out, src_v, dst_v, rows_a, rows_b, acc, sem_a, sem_b):
    cid = lax.axis_index("c")
    sid = lax.axis_index("s")
    base = sid * _RPT

    def zrow(i, carry):
        rows_a[lax.div(i, 8), pl.ds(lax.rem(i, 8) * 16, 16)] = jnp.zeros(
            (16,), jnp.float32)
        return carry

    lax.fori_loop(0, _CHUNK * 8, zrow, 0)
    for r in range(_RPT // _CHUNK):
        pltpu.sync_copy(rows_a, acc.at[pl.ds(base + r * _CHUNK, _CHUNK)])
    plsc.subcore_barrier()

    def run_chunks(tile0, ngrp):
        for g in range(ngrp):
            goff = tile0 + g * _GRP
            pltpu.sync_copy(srcs.at[pl.ds(goff, _GRP)], src_v)
            pltpu.sync_copy(dsts.at[pl.ds(goff, _GRP)], dst_v)
            pltpu.async_copy(feat.at[src_v.at[0]], rows_a, sem_a)

            def chunk(j, carry):
                @pl.when(lax.rem(j, 2) == 0)
                def _():
                    @pl.when(j + 1 < _GRP)
                    def _():
                        pltpu.async_copy(feat.at[src_v.at[j + 1]], rows_b, sem_b)
                    pltpu.make_async_copy(feat.at[src_v.at[j]], rows_a, sem_a).wait()
                    pltpu.sync_copy(rows_a, acc.at[dst_v.at[j]], add=True)

                @pl.when(lax.rem(j, 2) == 1)
                def _():
                    @pl.when(j + 1 < _GRP)
                    def _():
                        pltpu.async_copy(feat.at[src_v.at[j + 1]], rows_a, sem_a)
                    pltpu.make_async_copy(feat.at[src_v.at[j]], rows_b, sem_b).wait()
                    pltpu.sync_copy(rows_b, acc.at[dst_v.at[j]], add=True)

                return carry

            lax.fori_loop(0, _GRP, chunk, 0)

    @pl.when(cid == 0)
    def _():
        run_chunks(sid * _K0, _K0 // _GRP)

    @pl.when(cid == 1)
    def _():
        run_chunks(_NS * _K0 + sid * _K1, _K1 // _GRP)

    plsc.subcore_barrier()
    pltpu.sync_copy(acc.at[pl.ds(base, _RPT)], out.at[cid, pl.ds(base, _RPT)])


def _dot_t(a, w):
    return lax.dot_general(a, w, (((1,), (1,)), ((), ())),
                           preferred_element_type=jnp.float32)


def _conv_body(agg_ref, feat_ref, wl_ref, bl_ref, wr_ref, out_ref):
    a = agg_ref[0] + agg_ref[1]
    h = _dot_t(a, wl_ref[...]) + bl_ref[...] + _dot_t(feat_ref[...], wr_ref[...])
    out_ref[...] = jnp.maximum(h, 0.0)


_conv_tc = pl.pallas_call(
    _conv_body,
    grid=(_RPAD // _BLK,),
    in_specs=[
        pl.BlockSpec((_NC, _BLK, _D), lambda i: (0, i, 0)),
        pl.BlockSpec((_BLK, _D), lambda i: (i, 0)),
        pl.BlockSpec((_D, _D), lambda i: (0, 0)),
        pl.BlockSpec((1, _D), lambda i: (0, 0)),
        pl.BlockSpec((_D, _D), lambda i: (0, 0)),
    ],
    out_specs=pl.BlockSpec((_BLK, _D), lambda i: (i, 0)),
    out_shape=jax.ShapeDtypeStruct((_RPAD, _D), jnp.float32),
)


def _conv_pool_body(agg_ref, feat_ref, batch_ref, wl_ref, bl_ref, wr_ref,
                    w1_ref, b1_ref, w2_ref, b2_ref, w3_ref, b3_ref,
                    out_ref, pooled):
    i = pl.program_id(0)
    a = agg_ref[0] + agg_ref[1]
    h = jnp.maximum(
        _dot_t(a, wl_ref[...]) + bl_ref[...] + _dot_t(feat_ref[...], wr_ref[...]),
        0.0,
    )
    bb = batch_ref[0, 0, :]
    onehot = (bb[None, :] == lax.broadcasted_iota(jnp.int32, (_G, _BLK), 0)
              ).astype(jnp.float32)
    contrib = jnp.dot(onehot, h, preferred_element_type=jnp.float32)

    @pl.when(i == 0)
    def _():
        pooled[...] = contrib

    @pl.when(i > 0)
    def _():
        pooled[...] = pooled[...] + contrib

    @pl.when(i == pl.num_programs(0) - 1)
    def _():
        z = jnp.maximum(_dot_t(pooled[...], w1_ref[...]) + b1_ref[...], 0.0)
        z = jnp.maximum(_dot_t(z, w2_ref[...]) + b2_ref[...], 0.0)
        out_ref[...] = _dot_t(z, w3_ref[...]) + b3_ref[...]


_conv_pool_tc = pl.pallas_call(
    _conv_pool_body,
    grid=(_RPAD // _BLK,),
    in_specs=[
        pl.BlockSpec((_NC, _BLK, _D), lambda i: (0, i, 0)),
        pl.BlockSpec((_BLK, _D), lambda i: (i, 0)),
        pl.BlockSpec((1, 1, _BLK), lambda i: (i, 0, 0)),
        pl.BlockSpec((_D, _D), lambda i: (0, 0)),
        pl.BlockSpec((1, _D), lambda i: (0, 0)),
        pl.BlockSpec((_D, _D), lambda i: (0, 0)),
        pl.BlockSpec((64, _D), lambda i: (0, 0)),
        pl.BlockSpec((1, 64), lambda i: (0, 0)),
        pl.BlockSpec((32, 64), lambda i: (0, 0)),
        pl.BlockSpec((1, 32), lambda i: (0, 0)),
        pl.BlockSpec((10, 32), lambda i: (0, 0)),
        pl.BlockSpec((1, 10), lambda i: (0, 0)),
    ],
    out_specs=pl.BlockSpec((_G, 10), lambda i: (0, 0)),
    out_shape=jax.ShapeDtypeStruct((_G, 10), jnp.float32),
    scratch_shapes=[pltpu.VMEM((_G, _D), jnp.float32)],
)


def kernel(x, edge_index, batch, Wl1, bl1, Wr1, Wl2, bl2, Wr2, W1, b1, W2, b2, W3, b3):
    src = edge_index[0]
    dst = edge_index[1]
    pad_e = _EPAD - _E
    pad_src = jnp.arange(pad_e, dtype=jnp.int32) % _N
    srcs = jnp.concatenate([src, pad_src]).reshape(_TOTCH, _CHUNK)
    pad_dst = _N + (jnp.arange(pad_e, dtype=jnp.int32) % (_RPAD - _N))
    dsts = jnp.concatenate([dst, pad_dst]).reshape(_TOTCH, _CHUNK)
    xp = jnp.pad(x, ((0, _RPAD - _N), (0, 0)))
    batch_r = jnp.pad(batch, (0, _RPAD - _N), constant_values=_G).reshape(
        _RPAD // _BLK, 1, _BLK)

    edge_agg = _edge_agg_kernel()
    agg1 = edge_agg(xp, srcs, dsts)
    h1 = _conv_tc(agg1, xp, Wl1, bl1.reshape(1, -1), Wr1)
    agg2 = edge_agg(h1, srcs, dsts)
    out = _conv_pool_tc(agg2, h1, batch_r, Wl2, bl2.reshape(1, -1), Wr2,
                        W1, b1.reshape(1, -1), W2, b2.reshape(1, -1),
                        W3, b3.reshape(1, -1))
    return out


# speedup vs baseline: 3.9279x; 1.0557x over previous
import functools

import jax
import jax.numpy as jnp
from jax import lax
from jax.experimental import pallas as pl
from jax.experimental.pallas import tpu as pltpu
from jax.experimental.pallas import tpu_sc as plsc

_N = 10000
_D = 128
_E = 320000
_G = 64

_NC = 2
_NS = 16
_CHUNK = 128
_TOTCH = 2560
_K0 = 80
_K1 = (_TOTCH // _NS) - _K0
_GRP = 40
_EPAD = _TOTCH * _CHUNK
_RPAD = 10240
_RPT = _RPAD // _NS
_BLK = 512


@functools.lru_cache(maxsize=None)
def _edge_agg_kernel():
    return functools.partial(
        pl.kernel,
        out_type=jax.ShapeDtypeStruct((_NC, _RPAD, _D), jnp.float32),
        mesh=plsc.VectorSubcoreMesh(
            core_axis_name="c", subcore_axis_name="s", num_cores=_NC, num_subcores=_NS
        ),
        scratch_types=[
            pltpu.VMEM((_GRP, _CHUNK), jnp.int32),
            pltpu.VMEM((_GRP, _CHUNK), jnp.int32),
            pltpu.VMEM((_CHUNK, _D), jnp.float32),
            pltpu.VMEM((_CHUNK, _D), jnp.float32),
            pltpu.VMEM_SHARED((_RPAD, _D), jnp.float32),
            pltpu.SemaphoreType.DMA,
            pltpu.SemaphoreType.DMA,
        ],
    )(_edge_agg_body)


def _edge_agg_body(feat, srcs, dsts, out, src_v, dst_v, rows_a, rows_b, acc, sem_a, sem_b):
    cid = lax.axis_index("c")
    sid = lax.axis_index("s")
    base = sid * _RPT

    def zrow(i, carry):
        rows_a[lax.div(i, 8), pl.ds(lax.rem(i, 8) * 16, 16)] = jnp.zeros(
            (16,), jnp.float32)
        return carry

    lax.fori_loop(0, _CHUNK * 8, zrow, 0)
    for r in range(_RPT // _CHUNK):
        pltpu.sync_copy(rows_a, acc.at[pl.ds(base + r * _CHUNK, _CHUNK)])
    plsc.subcore_barrier()

    def run_chunks(tile0, ngrp):
        for g in range(ngrp):
            goff = tile0 + g * _GRP
            pltpu.sync_copy(srcs.at[pl.ds(goff, _GRP)], src_v)
            pltpu.sync_copy(dsts.at[pl.ds(goff, _GRP)], dst_v)
            pltpu.async_copy(feat.at[src_v.at[0]], rows_a, sem_a)

            def chunk(j, carry):
                @pl.when(lax.rem(j, 2) == 0)
                def _():
                    @pl.when(j + 1 < _GRP)
                    def _():
                        pltpu.async_copy(feat.at[src_v.at[j + 1]], rows_b, sem_b)
                    pltpu.make_async_copy(feat.at[src_v.at[j]], rows_a, sem_a).wait()
                    pltpu.sync_copy(rows_a, acc.at[dst_v.at[j]], add=True)

                @pl.when(lax.rem(j, 2) == 1)
                def _():
                    @pl.when(j + 1 < _GRP)
                    def _():
                        pltpu.async_copy(feat.at[src_v.at[j + 1]], rows_a, sem_a)
                    pltpu.make_async_copy(feat.at[src_v.at[j]], rows_b, sem_b).wait()
                    pltpu.sync_copy(rows_b, acc.at[dst_v.at[j]], add=True)

                return carry

            lax.fori_loop(0, _GRP, chunk, 0)

    @pl.when(cid == 0)
    def _():
        run_chunks(sid * _K0, _K0 // _GRP)

    @pl.when(cid == 1)
    def _():
        run_chunks(_NS * _K0 + sid * _K1, _K1 // _GRP)

    plsc.subcore_barrier()
    pltpu.sync_copy(acc.at[pl.ds(base, _RPT)], out.at[cid, pl.ds(base, _RPT)])


def _dot_t(a, w):
    return lax.dot_general(a, w, (((1,), (1,)), ((), ())),
                           preferred_element_type=jnp.float32)


def _conv_body(agg_ref, feat_ref, wl_ref, bl_ref, wr_ref, out_ref):
    a = agg_ref[0] + agg_ref[1]
    h = _dot_t(a, wl_ref[...]) + bl_ref[...] + _dot_t(feat_ref[...], wr_ref[...])
    out_ref[...] = jnp.maximum(h, 0.0)


_conv_tc = pl.pallas_call(
    _conv_body,
    grid=(_RPAD // _BLK,),
    in_specs=[
        pl.BlockSpec((_NC, _BLK, _D), lambda i: (0, i, 0)),
        pl.BlockSpec((_BLK, _D), lambda i: (i, 0)),
        pl.BlockSpec((_D, _D), lambda i: (0, 0)),
        pl.BlockSpec((1, _D), lambda i: (0, 0)),
        pl.BlockSpec((_D, _D), lambda i: (0, 0)),
    ],
    out_specs=pl.BlockSpec((_BLK, _D), lambda i: (i, 0)),
    out_shape=jax.ShapeDtypeStruct((_RPAD, _D), jnp.float32),
)


def _conv_pool_body(agg_ref, feat_ref, batch_ref, wl_ref, bl_ref, wr_ref,
                    w1_ref, b1_ref, w2_ref, b2_ref, w3_ref, b3_ref,
                    out_ref, pooled):
    i = pl.program_id(0)
    a = agg_ref[0] + agg_ref[1]
    h = jnp.maximum(
        _dot_t(a, wl_ref[...]) + bl_ref[...] + _dot_t(feat_ref[...], wr_ref[...]),
        0.0,
    )
    bb = batch_ref[0, 0, :]
    onehot = (bb[None, :] == lax.broadcasted_iota(jnp.int32, (_G, _BLK), 0)
              ).astype(jnp.float32)
    contrib = jnp.dot(onehot, h, preferred_element_type=jnp.float32)

    @pl.when(i == 0)
    def _():
        pooled[...] = contrib

    @pl.when(i > 0)
    def _():
        pooled[...] = pooled[...] + contrib

    @pl.when(i == pl.num_programs(0) - 1)
    def _():
        z = jnp.maximum(_dot_t(pooled[...], w1_ref[...]) + b1_ref[...], 0.0)
        z = jnp.maximum(_dot_t(z, w2_ref[...]) + b2_ref[...], 0.0)
        out_ref[...] = _dot_t(z, w3_ref[...]) + b3_ref[...]


_conv_pool_tc = pl.pallas_call(
    _conv_pool_body,
    grid=(_RPAD // _BLK,),
    in_specs=[
        pl.BlockSpec((_NC, _BLK, _D), lambda i: (0, i, 0)),
        pl.BlockSpec((_BLK, _D), lambda i: (i, 0)),
        pl.BlockSpec((1, 1, _BLK), lambda i: (i, 0, 0)),
        pl.BlockSpec((_D, _D), lambda i: (0, 0)),
        pl.BlockSpec((1, _D), lambda i: (0, 0)),
        pl.BlockSpec((_D, _D), lambda i: (0, 0)),
        pl.BlockSpec((64, _D), lambda i: (0, 0)),
        pl.BlockSpec((1, 64), lambda i: (0, 0)),
        pl.BlockSpec((32, 64), lambda i: (0, 0)),
        pl.BlockSpec((1, 32), lambda i: (0, 0)),
        pl.BlockSpec((10, 32), lambda i: (0, 0)),
        pl.BlockSpec((1, 10), lambda i: (0, 0)),
    ],
    out_specs=pl.BlockSpec((_G, 10), lambda i: (0, 0)),
    out_shape=jax.ShapeDtypeStruct((_G, 10), jnp.float32),
    scratch_shapes=[pltpu.VMEM((_G, _D), jnp.float32)],
)


def kernel(x, edge_index, batch, Wl1, bl1, Wr1, Wl2, bl2, Wr2, W1, b1, W2, b2, W3, b3):
    src = edge_index[0]
    dst = edge_index[1]
    pad_e = _EPAD - _E
    pad_src = jnp.arange(pad_e, dtype=jnp.int32) % _N
    srcs = jnp.concatenate([src, pad_src]).reshape(_TOTCH, _CHUNK)
    pad_dst = _N + (jnp.arange(pad_e, dtype=jnp.int32) % (_RPAD - _N))
    dsts = jnp.concatenate([dst, pad_dst]).reshape(_TOTCH, _CHUNK)
    xp = jnp.pad(x, ((0, _RPAD - _N), (0, 0)))
    batch_r = jnp.pad(batch, (0, _RPAD - _N), constant_values=_G).reshape(
        _RPAD // _BLK, 1, _BLK)

    edge_agg = _edge_agg_kernel()
    agg1 = edge_agg(xp, srcs, dsts)
    h1 = _conv_tc(agg1, xp, Wl1, bl1.reshape(1, -1), Wr1)
    agg2 = edge_agg(h1, srcs, dsts)
    out = _conv_pool_tc(agg2, h1, batch_r, Wl2, bl2.reshape(1, -1), Wr2,
                        W1, b1.reshape(1, -1), W2, b2.reshape(1, -1),
                        W3, b3.reshape(1, -1))
    return out
